# baseline (speedup 1.0000x reference)
import functools

import jax
import jax.numpy as jnp
from jax import lax
from jax.experimental import pallas as pl
from jax.experimental.pallas import tpu as pltpu
from jax.experimental.pallas import tpu_sc as plsc

N_ATOMS = 262144
D_OUT = 366
D_EMB = 87
D_TOP = 88
T_STRIDE = 88
CA = 256
LANES = 16


@functools.lru_cache(maxsize=1)
def _build():
    info = plsc.get_sparse_core_info()
    nw = info.num_cores * info.num_subcores
    atoms_per_w = N_ATOMS // nw
    n_chunks = atoms_per_w // CA
    n_pairs = n_chunks // 2
    groups = CA // LANES
    d_bot = D_OUT - D_TOP

    mesh = plsc.VectorSubcoreMesh(core_axis_name="c", subcore_axis_name="s")

    @functools.partial(
        pl.kernel,
        mesh=mesh,
        compiler_params=pltpu.CompilerParams(needs_layout_passes=False),
        out_type=jax.ShapeDtypeStruct((D_OUT, N_ATOMS), jnp.float32),
        scratch_types=[
            pltpu.VMEM((D_EMB * T_STRIDE,), jnp.float32),
            pltpu.VMEM((CA,), jnp.int32),
            pltpu.VMEM((CA,), jnp.int32),
            pltpu.VMEM((D_TOP, CA), jnp.float32),
            pltpu.VMEM((D_TOP, CA), jnp.float32),
            pltpu.VMEM((d_bot, CA), jnp.float32),
            pltpu.SemaphoreType.DMA,
            pltpu.SemaphoreType.DMA,
            pltpu.SemaphoreType.DMA,
            pltpu.SemaphoreType.DMA,
        ],
    )
    def k(z_hbm, tableT_hbm, out_hbm, tab_v, zc0, zc1, buf0, buf1, zbuf,
          sem0, sem1, zsem0, zsem1):
        wid = lax.axis_index("s") * info.num_cores + lax.axis_index("c")
        col0 = wid * atoms_per_w

        pltpu.sync_copy(tableT_hbm, tab_v)

        zero16 = jnp.zeros((LANES,), jnp.float32)

        def zrow(d, _):
            for g in range(groups):
                zbuf[d, pl.ds(g * LANES, LANES)] = zero16
            return _

        lax.fori_loop(0, d_bot, zrow, 0, unroll=2)
        for g in range(groups):
            buf0[D_EMB, pl.ds(g * LANES, LANES)] = zero16
            buf1[D_EMB, pl.ds(g * LANES, LANES)] = zero16

        def fill(buf, zc, i):
            pltpu.sync_copy(z_hbm.at[pl.ds(col0 + i * CA, CA)], zc)
            zvs = [zc[pl.ds(g * LANES, LANES)] for g in range(groups)]

            def drow(d, _):
                base = d * T_STRIDE
                xs = [
                    plsc.load_gather(tab_v, [base + zvs[g]])
                    for g in range(groups)
                ]
                for g in range(groups):
                    buf[d, pl.ds(g * LANES, LANES)] = xs[g]
                return _

            lax.fori_loop(0, D_EMB, drow, 0, unroll=2)

        def start(buf, i, sem, zsem):
            top = pltpu.async_copy(
                buf,
                out_hbm.at[pl.ds(0, D_TOP), pl.ds(col0 + i * CA, CA)],
                sem,
            )
            bot = pltpu.async_copy(
                zbuf,
                out_hbm.at[pl.ds(D_TOP, d_bot), pl.ds(col0 + i * CA, CA)],
                zsem,
            )
            return top, bot

        def drain(buf, sem, zsem):
            pltpu.make_async_copy(
                buf, out_hbm.at[pl.ds(0, D_TOP), pl.ds(col0, CA)], sem
            ).wait()
            pltpu.make_async_copy(
                zbuf, out_hbm.at[pl.ds(D_TOP, d_bot), pl.ds(col0, CA)], zsem
            ).wait()

        def body(j, _):
            @pl.when(j > 0)
            def _w0():
                drain(buf0, sem0, zsem0)

            fill(buf0, zc0, 2 * j)
            start(buf0, 2 * j, sem0, zsem0)

            @pl.when(j > 0)
            def _w1():
                drain(buf1, sem1, zsem1)

            fill(buf1, zc1, 2 * j + 1)
            start(buf1, 2 * j + 1, sem1, zsem1)
            return _

        lax.fori_loop(0, n_pairs, body, 0)
        drain(buf0, sem0, zsem0)
        drain(buf1, sem1, zsem1)

    return k


def kernel(Z, emb_table):
    z_flat = Z.reshape(-1)
    tableT = (
        jnp.zeros((D_EMB, T_STRIDE), jnp.float32)
        .at[:, :D_EMB]
        .set(emb_table.T)
    )
    out_t = _build()(z_flat, tableT.reshape(-1))
    return out_t.T
